# baseline (speedup 1.0000x reference)
import jax
import jax.numpy as jnp
from jax.experimental import pallas as pl
from jax.experimental.pallas import tpu as pltpu

_NB = 8
_BN = 8192 // _NB
_NBATCH = 16
_TOK = 1024
_D = 256


_WIN = 2736


def _round_bf16(v):
    b = jax.lax.bitcast_convert_type(v, jnp.int32)
    r = (b + jnp.int32(0x7FFF) + ((b >> 16) & jnp.int32(1))) \
        & jnp.int32(-65536)
    return jax.lax.bitcast_convert_type(r, jnp.float32)


def _dist_argmin_kernel(x2_ref, e2_ref, x_ref, e_ref, idx_ref, wmin_ref,
                        widx_ref):
    n = pl.program_id(1)
    nb = pl.num_programs(1)
    bn = e_ref.shape[0]
    tok = x_ref.shape[2]
    m = jax.lax.dot_general(
        e_ref[...], x_ref[0],
        dimension_numbers=(((1,), (0,)), ((), ())),
        preferred_element_type=jnp.float32)
    s = jnp.maximum(x2_ref[0] + e2_ref[...] - 2.0 * m, 0.0)
    riota = jax.lax.broadcasted_iota(jnp.int32, (bn, tok), 0)
    gcode = riota + n * bn
    win = ((gcode >= _WIN).astype(jnp.int32)
           + (gcode >= 2 * _WIN).astype(jnp.int32))
    inf = jnp.float32(jnp.inf)

    @pl.when(n == 0)
    def _():
        wmin_ref[...] = jnp.full(wmin_ref.shape, inf, jnp.float32)
        widx_ref[...] = jnp.zeros(widx_ref.shape, jnp.int32)

    for w in range(3):
        sm = jnp.where(win == w, s, inf)
        lmin = jnp.min(sm, axis=0, keepdims=True)
        larg = jnp.min(jnp.where(sm == lmin, gcode, jnp.int32(2 ** 30)),
                       axis=0, keepdims=True)
        prev = wmin_ref[pl.ds(w, 1), :]
        take = lmin < prev
        wmin_ref[pl.ds(w, 1), :] = jnp.where(take, lmin, prev)
        widx_ref[pl.ds(w, 1), :] = jnp.where(take, larg,
                                             widx_ref[pl.ds(w, 1), :])

    @pl.when(n == nb - 1)
    def _():
        acc = _round_bf16(wmin_ref[pl.ds(0, 1), :])
        idx = widx_ref[pl.ds(0, 1), :]
        for w in (1, 2):
            v = wmin_ref[pl.ds(w, 1), :]
            take = v < acc
            acc = jnp.where(take, _round_bf16(v), acc)
            idx = jnp.where(take, widx_ref[pl.ds(w, 1), :], idx)
        idx_ref[0] = idx


def _emit_kernel(idxl_ref, idxs_ref, x_ref, e_ref, enc_ref, q_ref, loss_ref,
                 perp_ref, hist_ref, lsum_ref):
    b = pl.program_id(0)
    n = pl.program_id(1)
    nbat = pl.num_programs(0)
    nb = pl.num_programs(1)
    bn = e_ref.shape[0]
    tok = x_ref.shape[2]
    code0 = n * bn

    cols = jax.lax.broadcasted_iota(jnp.int32, (tok, bn), 1) + code0
    t_oh = (idxs_ref[0] == cols).astype(jnp.float32)
    enc_ref[...] = t_oh

    rows = jax.lax.broadcasted_iota(jnp.int32, (bn, tok), 0) + code0
    tt_oh = (idxl_ref[0] == rows).astype(jnp.float32)
    qpart = jax.lax.dot_general(
        e_ref[...], tt_oh,
        dimension_numbers=(((0,), (0,)), ((), ())),
        preferred_element_type=jnp.float32,
        precision=jax.lax.Precision.HIGHEST)

    @pl.when(n == 0)
    def _():
        q_ref[0] = qpart

    @pl.when(n > 0)
    def _():
        q_ref[0] = q_ref[0] + qpart

    colsum = jnp.sum(t_oh, axis=0, keepdims=True)

    @pl.when(b == 0)
    def _():
        hist_ref[pl.ds(n, 1), :] = colsum

    @pl.when(b > 0)
    def _():
        hist_ref[pl.ds(n, 1), :] = hist_ref[pl.ds(n, 1), :] + colsum

    @pl.when(n == nb - 1)
    def _():
        xb = x_ref[0]
        qfull = q_ref[0]
        diff = xb - qfull
        ss = jnp.sum(diff * diff)

        @pl.when(b == 0)
        def _():
            lsum_ref[0] = ss

        @pl.when(b > 0)
        def _():
            lsum_ref[0] = lsum_ref[0] + ss

        q_ref[0] = xb + (qfull - xb)

        @pl.when(b == nbat - 1)
        def _():
            loss_ref[0, 0] = 0.25 * (lsum_ref[0] / (16384.0 * 256.0))
            p = hist_ref[...] * (1.0 / 16384.0)
            plogp = p * jnp.log(p + 1e-10)
            perp_ref[0, 0] = jnp.exp(-jnp.sum(plogp))


def _argmin_call(x2in, e2in, xv, codebook):
    return pl.pallas_call(
        _dist_argmin_kernel,
        grid=(_NBATCH, _NB),
        in_specs=[
            pl.BlockSpec((1, 1, _TOK), lambda b, n: (b, 0, 0)),
            pl.BlockSpec((_BN, 1), lambda b, n: (n, 0)),
            pl.BlockSpec((1, _D, _TOK), lambda b, n: (b, 0, 0)),
            pl.BlockSpec((_BN, _D), lambda b, n: (n, 0)),
        ],
        out_specs=pl.BlockSpec((1, 1, _TOK), lambda b, n: (b, 0, 0)),
        out_shape=jax.ShapeDtypeStruct((_NBATCH, 1, _TOK), jnp.int32),
        scratch_shapes=[
            pltpu.VMEM((8, _TOK), jnp.float32),
            pltpu.VMEM((8, _TOK), jnp.int32),
        ],
        compiler_params=pltpu.CompilerParams(
            dimension_semantics=("arbitrary", "arbitrary")),
    )(x2in, e2in, xv, codebook)


def _argmin_only(inputs, codebook):
    xv = inputs.reshape(_NBATCH, _D, _TOK)
    flat = jax.lax.optimization_barrier(
        jnp.transpose(inputs, (0, 2, 3, 1)).reshape(-1, _D))
    x2 = jnp.sum(flat ** 2, axis=1)
    e2 = jnp.sum(codebook ** 2, axis=1)
    return _argmin_call(x2.reshape(_NBATCH, 1, _TOK), e2.reshape(8192, 1),
                        xv, codebook)


def _run(inputs, codebook):
    xv = inputs.reshape(_NBATCH, _D, _TOK)
    flat = jax.lax.optimization_barrier(
        jnp.transpose(inputs, (0, 2, 3, 1)).reshape(-1, _D))
    x2 = jnp.sum(flat ** 2, axis=1)
    e2 = jnp.sum(codebook ** 2, axis=1)
    x2in = x2.reshape(_NBATCH, 1, _TOK)
    e2in = e2.reshape(8192, 1)

    idx = _argmin_call(x2in, e2in, xv, codebook)

    idxs = idx.reshape(_NBATCH, _TOK, 1)

    enc, qv, loss2d, perp2d = pl.pallas_call(
        _emit_kernel,
        grid=(_NBATCH, _NB),
        in_specs=[
            pl.BlockSpec((1, 1, _TOK), lambda b, n: (b, 0, 0)),
            pl.BlockSpec((1, _TOK, 1), lambda b, n: (b, 0, 0)),
            pl.BlockSpec((1, _D, _TOK), lambda b, n: (b, 0, 0)),
            pl.BlockSpec((_BN, _D), lambda b, n: (n, 0)),
        ],
        out_specs=[
            pl.BlockSpec((_TOK, _BN), lambda b, n: (b, n)),
            pl.BlockSpec((1, _D, _TOK), lambda b, n: (b, 0, 0)),
            pl.BlockSpec(memory_space=pltpu.SMEM),
            pl.BlockSpec(memory_space=pltpu.SMEM),
        ],
        out_shape=[
            jax.ShapeDtypeStruct((_NBATCH * _TOK, 8192), jnp.float32),
            jax.ShapeDtypeStruct((_NBATCH, _D, _TOK), jnp.float32),
            jax.ShapeDtypeStruct((1, 1), jnp.float32),
            jax.ShapeDtypeStruct((1, 1), jnp.float32),
        ],
        scratch_shapes=[
            pltpu.VMEM((_NB, _BN), jnp.float32),
            pltpu.SMEM((1,), jnp.float32),
        ],
        compiler_params=pltpu.CompilerParams(
            dimension_semantics=("arbitrary", "arbitrary")),
    )(idx, idxs, xv, codebook)

    quantized = qv.reshape(_NBATCH, _D, 32, 32)
    return (quantized, loss2d.reshape(()), perp2d.reshape(()), enc)


def kernel(inputs, codebook):
    return _run(inputs, codebook)


# speedup vs baseline: 2.0465x; 2.0465x over previous
import functools

import jax
import jax.numpy as jnp
from jax.experimental import pallas as pl
from jax.experimental.pallas import tpu as pltpu
from jax.experimental.pallas import tpu_sc as plsc

_NB = 8
_BN = 8192 // _NB
_NBATCH = 16
_TOK = 1024
_D = 256

_WIN = 2736


def _segments(k):
    segs = []
    blk_lo, blk_hi = k * _BN, (k + 1) * _BN
    for w in range(3):
        w_lo, w_hi = w * _WIN, min((w + 1) * _WIN, 8192)
        lo, hi = max(blk_lo, w_lo), min(blk_hi, w_hi)
        if lo < hi:
            segs.append((lo - blk_lo, hi - blk_lo, w))
    return segs


def _round_bf16(v):
    b = jax.lax.bitcast_convert_type(v, jnp.int32)
    r = (b + jnp.int32(0x7FFF) + ((b >> 16) & jnp.int32(1))) \
        & jnp.int32(-65536)
    return jax.lax.bitcast_convert_type(r, jnp.float32)


def _dist_argmin_kernel(x2_ref, e2_ref, x_ref, e_ref, idx_ref, wmin_ref,
                        widx_ref):
    n = pl.program_id(1)
    nb = pl.num_programs(1)
    bn = e_ref.shape[0]
    tok = x_ref.shape[2]
    m = jax.lax.dot_general(
        e_ref[...], x_ref[0],
        dimension_numbers=(((1,), (0,)), ((), ())),
        preferred_element_type=jnp.float32)
    s = jnp.maximum(x2_ref[0] + e2_ref[...] - 2.0 * m, 0.0)
    riota = jax.lax.broadcasted_iota(jnp.int32, (bn, tok), 0)
    inf = jnp.float32(jnp.inf)

    @pl.when(n == 0)
    def _():
        wmin_ref[...] = jnp.full(wmin_ref.shape, inf, jnp.float32)
        widx_ref[...] = jnp.zeros(widx_ref.shape, jnp.int32)

    for k in range(_NB):
        @pl.when(n == k)
        def _(k=k):
            for (lo, hi, w) in _segments(k):
                if lo == 0 and hi == bn:
                    sm = s
                else:
                    sm = jnp.where((riota >= lo) & (riota < hi), s, inf)
                lmin = jnp.min(sm, axis=0, keepdims=True)
                larg = jnp.min(
                    jnp.where(sm == lmin, riota, jnp.int32(2 ** 30)),
                    axis=0, keepdims=True) + k * bn
                prev = wmin_ref[pl.ds(w, 1), :]
                take = lmin < prev
                wmin_ref[pl.ds(w, 1), :] = jnp.where(take, lmin, prev)
                widx_ref[pl.ds(w, 1), :] = jnp.where(
                    take, larg, widx_ref[pl.ds(w, 1), :])

    @pl.when(n == nb - 1)
    def _():
        acc = _round_bf16(wmin_ref[pl.ds(0, 1), :])
        idx = widx_ref[pl.ds(0, 1), :]
        for w in (1, 2):
            v = wmin_ref[pl.ds(w, 1), :]
            take = v < acc
            acc = jnp.where(take, _round_bf16(v), acc)
            idx = jnp.where(take, widx_ref[pl.ds(w, 1), :], idx)
        idx_ref[0] = idx


def _encode_kernel(idxs_ref, enc_ref, perp_ref, hist_ref):
    b = pl.program_id(0)
    n = pl.program_id(1)
    nbat = pl.num_programs(0)
    nb = pl.num_programs(1)
    tok, bn = enc_ref.shape
    cols = jax.lax.broadcasted_iota(jnp.int32, (tok, bn), 1) + n * bn
    t_oh = (idxs_ref[0] == cols).astype(jnp.float32)
    enc_ref[...] = t_oh
    colsum = jnp.sum(t_oh, axis=0, keepdims=True)

    @pl.when(b == 0)
    def _():
        hist_ref[pl.ds(n, 1), :] = colsum

    @pl.when(b > 0)
    def _():
        hist_ref[pl.ds(n, 1), :] = hist_ref[pl.ds(n, 1), :] + colsum

    @pl.when((b == nbat - 1) & (n == nb - 1))
    def _():
        p = hist_ref[...] * (1.0 / 16384.0)
        plogp = p * jnp.log(p + 1e-10)
        perp_ref[0, 0] = jnp.exp(-jnp.sum(plogp))


def _ste_kernel(q_ref, x_ref, out_ref, loss_ref, lsum_ref):
    b = pl.program_id(0)
    nbat = pl.num_programs(0)
    qt = jnp.transpose(q_ref[0], (1, 0))
    xb = x_ref[0]
    diff = xb - qt
    ss = jnp.sum(diff * diff)

    @pl.when(b == 0)
    def _():
        lsum_ref[0] = ss

    @pl.when(b > 0)
    def _():
        lsum_ref[0] = lsum_ref[0] + ss

    out_ref[0] = xb + (qt - xb)

    @pl.when(b == nbat - 1)
    def _():
        loss_ref[0, 0] = 0.25 * (lsum_ref[0] / (16384.0 * 256.0))


_SC_NC = 2
_SC_NS = 16
_SC_CH = 128


def _sc_gather(codebook, idx_flat):
    nw = _SC_NC * _SC_NS
    bpw = 16384 // nw
    mesh = plsc.VectorSubcoreMesh(core_axis_name="c", subcore_axis_name="s")

    @functools.partial(
        pl.kernel, mesh=mesh,
        out_type=jax.ShapeDtypeStruct((16384, _D), jnp.float32),
        scratch_types=[
            pltpu.VMEM((_SC_CH,), jnp.int32),
            pltpu.VMEM((_SC_CH, _D), jnp.float32),
            pltpu.SemaphoreType.DMA,
        ],
    )
    def k(table_hbm, idx_hbm, out_hbm, idx_v, rows_v, sem):
        wid = jax.lax.axis_index("s") * _SC_NC + jax.lax.axis_index("c")
        base = wid * bpw
        for c in range(bpw // _SC_CH):
            off = base + c * _SC_CH
            pltpu.sync_copy(idx_hbm.at[pl.ds(off, _SC_CH)], idx_v)
            pltpu.async_copy(table_hbm.at[idx_v], rows_v, sem).wait()
            pltpu.sync_copy(rows_v, out_hbm.at[pl.ds(off, _SC_CH)])

    return k(codebook, idx_flat)


def _argmin_call(x2in, e2in, xv, codebook):
    return pl.pallas_call(
        _dist_argmin_kernel,
        grid=(_NBATCH, _NB),
        in_specs=[
            pl.BlockSpec((1, 1, _TOK), lambda b, n: (b, 0, 0)),
            pl.BlockSpec((_BN, 1), lambda b, n: (n, 0)),
            pl.BlockSpec((1, _D, _TOK), lambda b, n: (b, 0, 0)),
            pl.BlockSpec((_BN, _D), lambda b, n: (n, 0)),
        ],
        out_specs=pl.BlockSpec((1, 1, _TOK), lambda b, n: (b, 0, 0)),
        out_shape=jax.ShapeDtypeStruct((_NBATCH, 1, _TOK), jnp.int32),
        scratch_shapes=[
            pltpu.VMEM((8, _TOK), jnp.float32),
            pltpu.VMEM((8, _TOK), jnp.int32),
        ],
        compiler_params=pltpu.CompilerParams(
            dimension_semantics=("arbitrary", "arbitrary")),
    )(x2in, e2in, xv, codebook)


def _argmin_only(inputs, codebook):
    xv = inputs.reshape(_NBATCH, _D, _TOK)
    flat = jax.lax.optimization_barrier(
        jnp.transpose(inputs, (0, 2, 3, 1)).reshape(-1, _D))
    x2 = jnp.sum(flat ** 2, axis=1)
    e2 = jnp.sum(codebook ** 2, axis=1)
    return _argmin_call(x2.reshape(_NBATCH, 1, _TOK), e2.reshape(8192, 1),
                        xv, codebook)


def _run(inputs, codebook):
    xv = inputs.reshape(_NBATCH, _D, _TOK)
    flat = jax.lax.optimization_barrier(
        jnp.transpose(inputs, (0, 2, 3, 1)).reshape(-1, _D))
    x2 = jnp.sum(flat ** 2, axis=1)
    e2 = jnp.sum(codebook ** 2, axis=1)
    x2in = x2.reshape(_NBATCH, 1, _TOK)
    e2in = e2.reshape(8192, 1)

    idx = _argmin_call(x2in, e2in, xv, codebook)

    idxs = idx.reshape(_NBATCH, _TOK, 1)
    enc, perp2d = pl.pallas_call(
        _encode_kernel,
        grid=(_NBATCH, _NB),
        in_specs=[
            pl.BlockSpec((1, _TOK, 1), lambda b, n: (b, 0, 0)),
        ],
        out_specs=[
            pl.BlockSpec((_TOK, _BN), lambda b, n: (b, n)),
            pl.BlockSpec(memory_space=pltpu.SMEM),
        ],
        out_shape=[
            jax.ShapeDtypeStruct((_NBATCH * _TOK, 8192), jnp.float32),
            jax.ShapeDtypeStruct((1, 1), jnp.float32),
        ],
        scratch_shapes=[
            pltpu.VMEM((_NB, _BN), jnp.float32),
        ],
        compiler_params=pltpu.CompilerParams(
            dimension_semantics=("arbitrary", "arbitrary")),
    )(idxs)

    qf = _sc_gather(codebook, idx.reshape(16384))
    qn = qf.reshape(_NBATCH, _TOK, _D)

    qv, loss2d = pl.pallas_call(
        _ste_kernel,
        grid=(_NBATCH,),
        in_specs=[
            pl.BlockSpec((1, _TOK, _D), lambda b: (b, 0, 0)),
            pl.BlockSpec((1, _D, _TOK), lambda b: (b, 0, 0)),
        ],
        out_specs=[
            pl.BlockSpec((1, _D, _TOK), lambda b: (b, 0, 0)),
            pl.BlockSpec(memory_space=pltpu.SMEM),
        ],
        out_shape=[
            jax.ShapeDtypeStruct((_NBATCH, _D, _TOK), jnp.float32),
            jax.ShapeDtypeStruct((1, 1), jnp.float32),
        ],
        scratch_shapes=[
            pltpu.SMEM((1,), jnp.float32),
        ],
        compiler_params=pltpu.CompilerParams(
            dimension_semantics=("arbitrary",)),
    )(qn, xv)

    quantized = qv.reshape(_NBATCH, _D, 32, 32)
    return (quantized, loss2d.reshape(()), perp2d.reshape(()), enc)


def kernel(inputs, codebook):
    return _run(inputs, codebook)


# speedup vs baseline: 2.0500x; 1.0017x over previous
import functools

import jax
import jax.numpy as jnp
from jax.experimental import pallas as pl
from jax.experimental.pallas import tpu as pltpu
from jax.experimental.pallas import tpu_sc as plsc

_NB = 8
_BN = 8192 // _NB
_NBATCH = 16
_TOK = 1024
_D = 256

_WIN = 2736


def _segments(k):
    segs = []
    blk_lo, blk_hi = k * _BN, (k + 1) * _BN
    for w in range(3):
        w_lo, w_hi = w * _WIN, min((w + 1) * _WIN, 8192)
        lo, hi = max(blk_lo, w_lo), min(blk_hi, w_hi)
        if lo < hi:
            segs.append((lo - blk_lo, hi - blk_lo, w))
    return segs


def _round_bf16(v):
    b = jax.lax.bitcast_convert_type(v, jnp.int32)
    r = (b + jnp.int32(0x7FFF) + ((b >> 16) & jnp.int32(1))) \
        & jnp.int32(-65536)
    return jax.lax.bitcast_convert_type(r, jnp.float32)


def _dist_argmin_kernel(x2_ref, e2_ref, x_ref, e_ref, idx_ref, wmin_ref,
                        widx_ref):
    n = pl.program_id(1)
    nb = pl.num_programs(1)
    bn = e_ref.shape[0]
    tok = x_ref.shape[2]
    m = jax.lax.dot_general(
        e_ref[...], x_ref[0],
        dimension_numbers=(((1,), (0,)), ((), ())),
        preferred_element_type=jnp.float32)
    s = jnp.maximum(x2_ref[0] + e2_ref[...] - 2.0 * m, 0.0)
    riota = jax.lax.broadcasted_iota(jnp.int32, (bn, tok), 0)
    inf = jnp.float32(jnp.inf)

    @pl.when(n == 0)
    def _():
        wmin_ref[...] = jnp.full(wmin_ref.shape, inf, jnp.float32)
        widx_ref[...] = jnp.zeros(widx_ref.shape, jnp.int32)

    for k in range(_NB):
        @pl.when(n == k)
        def _(k=k):
            for (lo, hi, w) in _segments(k):
                if lo == 0 and hi == bn:
                    sm = s
                else:
                    sm = jnp.where((riota >= lo) & (riota < hi), s, inf)
                lmin = jnp.min(sm, axis=0, keepdims=True)
                larg = jnp.min(
                    jnp.where(sm == lmin, riota, jnp.int32(2 ** 30)),
                    axis=0, keepdims=True) + k * bn
                prev = wmin_ref[pl.ds(w, 1), :]
                take = lmin < prev
                wmin_ref[pl.ds(w, 1), :] = jnp.where(take, lmin, prev)
                widx_ref[pl.ds(w, 1), :] = jnp.where(
                    take, larg, widx_ref[pl.ds(w, 1), :])

    @pl.when(n == nb - 1)
    def _():
        acc = _round_bf16(wmin_ref[pl.ds(0, 1), :])
        idx = widx_ref[pl.ds(0, 1), :]
        for w in (1, 2):
            v = wmin_ref[pl.ds(w, 1), :]
            take = v < acc
            acc = jnp.where(take, _round_bf16(v), acc)
            idx = jnp.where(take, widx_ref[pl.ds(w, 1), :], idx)
        idx_ref[0] = idx


def _encode_kernel(idxs_ref, enc_ref, perp_ref, hist_ref):
    b = pl.program_id(0)
    n = pl.program_id(1)
    nbat = pl.num_programs(0)
    nb = pl.num_programs(1)
    tok, bn = enc_ref.shape
    cols = jax.lax.broadcasted_iota(jnp.int32, (tok, bn), 1) + n * bn
    t_oh = (idxs_ref[0] == cols).astype(jnp.float32)
    enc_ref[...] = t_oh
    colsum = jnp.sum(t_oh, axis=0, keepdims=True)

    @pl.when(b == 0)
    def _():
        hist_ref[pl.ds(n, 1), :] = colsum

    @pl.when(b > 0)
    def _():
        hist_ref[pl.ds(n, 1), :] = hist_ref[pl.ds(n, 1), :] + colsum

    @pl.when((b == nbat - 1) & (n == nb - 1))
    def _():
        p = hist_ref[...] * (1.0 / 16384.0)
        plogp = p * jnp.log(p + 1e-10)
        perp_ref[0, 0] = jnp.exp(-jnp.sum(plogp))


def _ste_kernel(q_ref, x_ref, out_ref, loss_ref, lsum_ref):
    b = pl.program_id(0)
    nbat = pl.num_programs(0)
    qt = jnp.transpose(q_ref[0], (1, 0))
    xb = x_ref[0]
    diff = xb - qt
    ss = jnp.sum(diff * diff)

    @pl.when(b == 0)
    def _():
        lsum_ref[0] = ss

    @pl.when(b > 0)
    def _():
        lsum_ref[0] = lsum_ref[0] + ss

    out_ref[0] = xb + (qt - xb)

    @pl.when(b == nbat - 1)
    def _():
        loss_ref[0, 0] = 0.25 * (lsum_ref[0] / (16384.0 * 256.0))


_SC_NC = 2
_SC_NS = 16
_SC_CH = 128


def _sc_gather(codebook, idx_flat):
    nw = _SC_NC * _SC_NS
    bpw = 16384 // nw
    mesh = plsc.VectorSubcoreMesh(core_axis_name="c", subcore_axis_name="s")

    @functools.partial(
        pl.kernel, mesh=mesh,
        out_type=jax.ShapeDtypeStruct((16384, _D), jnp.float32),
        scratch_types=[
            pltpu.VMEM((_SC_CH,), jnp.int32),
            pltpu.VMEM((_SC_CH, _D), jnp.float32),
            pltpu.SemaphoreType.DMA,
        ],
    )
    def k(table_hbm, idx_hbm, out_hbm, idx_v, rows_v, sem):
        wid = jax.lax.axis_index("s") * _SC_NC + jax.lax.axis_index("c")
        base = wid * bpw
        for c in range(bpw // _SC_CH):
            off = base + c * _SC_CH
            pltpu.sync_copy(idx_hbm.at[pl.ds(off, _SC_CH)], idx_v)
            pltpu.async_copy(table_hbm.at[idx_v], rows_v, sem).wait()
            pltpu.sync_copy(rows_v, out_hbm.at[pl.ds(off, _SC_CH)])

    return k(codebook, idx_flat)


def _argmin_call(x2in, e2in, xv, codebook):
    return pl.pallas_call(
        _dist_argmin_kernel,
        grid=(_NBATCH, _NB),
        in_specs=[
            pl.BlockSpec((1, 1, _TOK), lambda b, n: (b, 0, 0)),
            pl.BlockSpec((_BN, 1), lambda b, n: (n, 0)),
            pl.BlockSpec((1, _D, _TOK), lambda b, n: (b, 0, 0)),
            pl.BlockSpec((_BN, _D), lambda b, n: (n, 0)),
        ],
        out_specs=pl.BlockSpec((1, 1, _TOK), lambda b, n: (b, 0, 0)),
        out_shape=jax.ShapeDtypeStruct((_NBATCH, 1, _TOK), jnp.int32),
        scratch_shapes=[
            pltpu.VMEM((8, _TOK), jnp.float32),
            pltpu.VMEM((8, _TOK), jnp.int32),
        ],
        compiler_params=pltpu.CompilerParams(
            dimension_semantics=("arbitrary", "arbitrary")),
    )(x2in, e2in, xv, codebook)


def _argmin_only(inputs, codebook):
    xv = inputs.reshape(_NBATCH, _D, _TOK)
    flat = jax.lax.optimization_barrier(
        jnp.transpose(inputs, (0, 2, 3, 1)).reshape(-1, _D))
    x2 = jnp.sum(flat ** 2, axis=1)
    e2 = jnp.sum(codebook ** 2, axis=1)
    return _argmin_call(x2.reshape(_NBATCH, 1, _TOK), e2.reshape(8192, 1),
                        xv, codebook)


def _run(inputs, codebook):
    xv = inputs.reshape(_NBATCH, _D, _TOK)
    flat = jax.lax.optimization_barrier(
        jnp.transpose(inputs, (0, 2, 3, 1)).reshape(-1, _D))
    x2 = jnp.sum(flat ** 2, axis=1)
    e2 = jnp.sum(codebook ** 2, axis=1)
    x2in = x2.reshape(_NBATCH, 1, _TOK)
    e2in = e2.reshape(8192, 1)

    idx = _argmin_call(x2in, e2in, xv, codebook)

    qf = _sc_gather(codebook, idx.reshape(16384))
    qn = qf.reshape(_NBATCH, _TOK, _D)

    idxs = idx.reshape(_NBATCH, _TOK, 1)
    enc, perp2d = pl.pallas_call(
        _encode_kernel,
        grid=(_NBATCH, _NB),
        in_specs=[
            pl.BlockSpec((1, _TOK, 1), lambda b, n: (b, 0, 0)),
        ],
        out_specs=[
            pl.BlockSpec((_TOK, _BN), lambda b, n: (b, n)),
            pl.BlockSpec(memory_space=pltpu.SMEM),
        ],
        out_shape=[
            jax.ShapeDtypeStruct((_NBATCH * _TOK, 8192), jnp.float32),
            jax.ShapeDtypeStruct((1, 1), jnp.float32),
        ],
        scratch_shapes=[
            pltpu.VMEM((_NB, _BN), jnp.float32),
        ],
        compiler_params=pltpu.CompilerParams(
            dimension_semantics=("arbitrary", "arbitrary")),
    )(idxs)

    qv, loss2d = pl.pallas_call(
        _ste_kernel,
        grid=(_NBATCH,),
        in_specs=[
            pl.BlockSpec((1, _TOK, _D), lambda b: (b, 0, 0)),
            pl.BlockSpec((1, _D, _TOK), lambda b: (b, 0, 0)),
        ],
        out_specs=[
            pl.BlockSpec((1, _D, _TOK), lambda b: (b, 0, 0)),
            pl.BlockSpec(memory_space=pltpu.SMEM),
        ],
        out_shape=[
            jax.ShapeDtypeStruct((_NBATCH, _D, _TOK), jnp.float32),
            jax.ShapeDtypeStruct((1, 1), jnp.float32),
        ],
        scratch_shapes=[
            pltpu.SMEM((1,), jnp.float32),
        ],
        compiler_params=pltpu.CompilerParams(
            dimension_semantics=("arbitrary",)),
    )(qn, xv)

    quantized = qv.reshape(_NBATCH, _D, 32, 32)
    return (quantized, loss2d.reshape(()), perp2d.reshape(()), enc)


def kernel(inputs, codebook):
    return _run(inputs, codebook)


# speedup vs baseline: 2.1419x; 1.0448x over previous
import functools

import jax
import jax.numpy as jnp
from jax.experimental import pallas as pl
from jax.experimental.pallas import tpu as pltpu
from jax.experimental.pallas import tpu_sc as plsc

_NB = 8
_BN = 8192 // _NB
_NBATCH = 16
_TOK = 1024
_D = 256

_WIN = 2736


def _segments(k):
    segs = []
    blk_lo, blk_hi = k * _BN, (k + 1) * _BN
    for w in range(3):
        w_lo, w_hi = w * _WIN, min((w + 1) * _WIN, 8192)
        lo, hi = max(blk_lo, w_lo), min(blk_hi, w_hi)
        if lo < hi:
            segs.append((lo - blk_lo, hi - blk_lo, w))
    return segs


def _round_bf16(v):
    b = jax.lax.bitcast_convert_type(v, jnp.int32)
    r = (b + jnp.int32(0x7FFF) + ((b >> 16) & jnp.int32(1))) \
        & jnp.int32(-65536)
    return jax.lax.bitcast_convert_type(r, jnp.float32)


def _dist_argmin_kernel(x2_ref, e2_ref, x_ref, em2_ref, idx_ref, wmin_ref,
                        widx_ref):
    n = pl.program_id(1)
    bn = em2_ref.shape[0]
    tok = x_ref.shape[2]
    nb = pl.num_programs(1)
    d = jax.lax.dot_general(
        em2_ref[...], x_ref[0],
        dimension_numbers=(((1,), (0,)), ((), ())),
        preferred_element_type=jnp.float32)
    s = jnp.maximum((x2_ref[0] + e2_ref[...]) + d, 0.0)
    riota = jax.lax.broadcasted_iota(jnp.int32, (bn, tok), 0)
    inf = jnp.float32(jnp.inf)

    @pl.when(n == 0)
    def _():
        wmin_ref[...] = jnp.full(wmin_ref.shape, inf, jnp.float32)
        widx_ref[...] = jnp.zeros(widx_ref.shape, jnp.int32)

    for k in range(_NB):
        @pl.when(n == k)
        def _(k=k):
            for (lo, hi, w) in _segments(k):
                sm = s[lo:hi]
                lmin = jnp.min(sm, axis=0, keepdims=True)
                larg = jnp.min(
                    jnp.where(sm == lmin, riota[lo:hi], jnp.int32(2 ** 30)),
                    axis=0, keepdims=True) + k * bn
                prev = wmin_ref[pl.ds(w, 1), :]
                take = lmin < prev
                wmin_ref[pl.ds(w, 1), :] = jnp.where(take, lmin, prev)
                widx_ref[pl.ds(w, 1), :] = jnp.where(
                    take, larg, widx_ref[pl.ds(w, 1), :])

    @pl.when(n == nb - 1)
    def _():
        acc = _round_bf16(wmin_ref[pl.ds(0, 1), :])
        idx = widx_ref[pl.ds(0, 1), :]
        for w in (1, 2):
            v = wmin_ref[pl.ds(w, 1), :]
            take = v < acc
            acc = jnp.where(take, _round_bf16(v), acc)
            idx = jnp.where(take, widx_ref[pl.ds(w, 1), :], idx)
        idx_ref[0] = idx


def _encode_kernel(idxs_ref, enc_ref, perp_ref, hist_ref):
    b = pl.program_id(0)
    n = pl.program_id(1)
    nbat = pl.num_programs(0)
    nb = pl.num_programs(1)
    tok, bn = enc_ref.shape
    cols = jax.lax.broadcasted_iota(jnp.int32, (tok, bn), 1) + n * bn
    t_oh = (idxs_ref[0] == cols).astype(jnp.float32)
    enc_ref[...] = t_oh
    colsum = jnp.sum(t_oh, axis=0, keepdims=True)

    @pl.when(b == 0)
    def _():
        hist_ref[pl.ds(n, 1), :] = colsum

    @pl.when(b > 0)
    def _():
        hist_ref[pl.ds(n, 1), :] = hist_ref[pl.ds(n, 1), :] + colsum

    @pl.when((b == nbat - 1) & (n == nb - 1))
    def _():
        p = hist_ref[...] * (1.0 / 16384.0)
        plogp = p * jnp.log(p + 1e-10)
        perp_ref[0, 0] = jnp.exp(-jnp.sum(plogp))


def _ste_kernel(q_ref, x_ref, out_ref, loss_ref, lsum_ref):
    b = pl.program_id(0)
    nbat = pl.num_programs(0)
    qt = jnp.transpose(q_ref[0], (1, 0))
    xb = x_ref[0]
    diff = xb - qt
    ss = jnp.sum(diff * diff)

    @pl.when(b == 0)
    def _():
        lsum_ref[0] = ss

    @pl.when(b > 0)
    def _():
        lsum_ref[0] = lsum_ref[0] + ss

    out_ref[0] = xb + (qt - xb)

    @pl.when(b == nbat - 1)
    def _():
        loss_ref[0, 0] = 0.25 * (lsum_ref[0] / (16384.0 * 256.0))


_SC_NC = 2
_SC_NS = 16
_SC_CH = 128


def _sc_gather(codebook, idx_flat):
    nw = _SC_NC * _SC_NS
    bpw = 16384 // nw
    mesh = plsc.VectorSubcoreMesh(core_axis_name="c", subcore_axis_name="s")

    @functools.partial(
        pl.kernel, mesh=mesh,
        out_type=jax.ShapeDtypeStruct((16384, _D), jnp.float32),
        scratch_types=[
            pltpu.VMEM((_SC_CH,), jnp.int32),
            pltpu.VMEM((_SC_CH, _D), jnp.float32),
            pltpu.SemaphoreType.DMA,
        ],
    )
    def k(table_hbm, idx_hbm, out_hbm, idx_v, rows_v, sem):
        wid = jax.lax.axis_index("s") * _SC_NC + jax.lax.axis_index("c")
        base = wid * bpw
        for c in range(bpw // _SC_CH):
            off = base + c * _SC_CH
            pltpu.sync_copy(idx_hbm.at[pl.ds(off, _SC_CH)], idx_v)
            pltpu.async_copy(table_hbm.at[idx_v], rows_v, sem).wait()
            pltpu.sync_copy(rows_v, out_hbm.at[pl.ds(off, _SC_CH)])

    return k(codebook, idx_flat)


def _argmin_call(x2in, e2in, xv, em2):
    return pl.pallas_call(
        _dist_argmin_kernel,
        grid=(_NBATCH, _NB),
        in_specs=[
            pl.BlockSpec((1, 1, _TOK), lambda b, n: (b, 0, 0)),
            pl.BlockSpec((_BN, 1), lambda b, n: (n, 0)),
            pl.BlockSpec((1, _D, _TOK), lambda b, n: (b, 0, 0)),
            pl.BlockSpec((_BN, _D), lambda b, n: (n, 0)),
        ],
        out_specs=pl.BlockSpec((1, 1, _TOK), lambda b, n: (b, 0, 0)),
        out_shape=jax.ShapeDtypeStruct((_NBATCH, 1, _TOK), jnp.int32),
        scratch_shapes=[
            pltpu.VMEM((8, _TOK), jnp.float32),
            pltpu.VMEM((8, _TOK), jnp.int32),
        ],
        compiler_params=pltpu.CompilerParams(
            dimension_semantics=("arbitrary", "arbitrary")),
    )(x2in, e2in, xv, em2)


def _argmin_only(inputs, codebook):
    xv = inputs.reshape(_NBATCH, _D, _TOK)
    flat = jax.lax.optimization_barrier(
        jnp.transpose(inputs, (0, 2, 3, 1)).reshape(-1, _D))
    x2 = jnp.sum(flat ** 2, axis=1)
    e2 = jnp.sum(codebook ** 2, axis=1)
    return _argmin_call(x2.reshape(_NBATCH, 1, _TOK), e2.reshape(8192, 1),
                        xv, -2.0 * codebook)


def _run(inputs, codebook):
    xv = inputs.reshape(_NBATCH, _D, _TOK)
    flat = jax.lax.optimization_barrier(
        jnp.transpose(inputs, (0, 2, 3, 1)).reshape(-1, _D))
    x2 = jnp.sum(flat ** 2, axis=1)
    e2 = jnp.sum(codebook ** 2, axis=1)
    x2in = x2.reshape(_NBATCH, 1, _TOK)
    e2in = e2.reshape(8192, 1)

    idx = _argmin_call(x2in, e2in, xv, -2.0 * codebook)

    qf = _sc_gather(codebook, idx.reshape(16384))
    qn = qf.reshape(_NBATCH, _TOK, _D)

    idxs = idx.reshape(_NBATCH, _TOK, 1)
    enc, perp2d = pl.pallas_call(
        _encode_kernel,
        grid=(_NBATCH, _NB),
        in_specs=[
            pl.BlockSpec((1, _TOK, 1), lambda b, n: (b, 0, 0)),
        ],
        out_specs=[
            pl.BlockSpec((_TOK, _BN), lambda b, n: (b, n)),
            pl.BlockSpec(memory_space=pltpu.SMEM),
        ],
        out_shape=[
            jax.ShapeDtypeStruct((_NBATCH * _TOK, 8192), jnp.float32),
            jax.ShapeDtypeStruct((1, 1), jnp.float32),
        ],
        scratch_shapes=[
            pltpu.VMEM((_NB, _BN), jnp.float32),
        ],
        compiler_params=pltpu.CompilerParams(
            dimension_semantics=("arbitrary", "arbitrary")),
    )(idxs)

    qv, loss2d = pl.pallas_call(
        _ste_kernel,
        grid=(_NBATCH,),
        in_specs=[
            pl.BlockSpec((1, _TOK, _D), lambda b: (b, 0, 0)),
            pl.BlockSpec((1, _D, _TOK), lambda b: (b, 0, 0)),
        ],
        out_specs=[
            pl.BlockSpec((1, _D, _TOK), lambda b: (b, 0, 0)),
            pl.BlockSpec(memory_space=pltpu.SMEM),
        ],
        out_shape=[
            jax.ShapeDtypeStruct((_NBATCH, _D, _TOK), jnp.float32),
            jax.ShapeDtypeStruct((1, 1), jnp.float32),
        ],
        scratch_shapes=[
            pltpu.SMEM((1,), jnp.float32),
        ],
        compiler_params=pltpu.CompilerParams(
            dimension_semantics=("arbitrary",)),
    )(qn, xv)

    quantized = qv.reshape(_NBATCH, _D, 32, 32)
    return (quantized, loss2d.reshape(()), perp2d.reshape(()), enc)


def kernel(inputs, codebook):
    return _run(inputs, codebook)


# speedup vs baseline: 2.1539x; 1.0056x over previous
import functools

import jax
import jax.numpy as jnp
from jax.experimental import pallas as pl
from jax.experimental.pallas import tpu as pltpu
from jax.experimental.pallas import tpu_sc as plsc

_NB = 8
_BN = 8192 // _NB
_NBATCH = 16
_TOK = 1024
_D = 256

_WIN = 2736


def _segments(k):
    segs = []
    blk_lo, blk_hi = k * _BN, (k + 1) * _BN
    for w in range(3):
        w_lo, w_hi = w * _WIN, min((w + 1) * _WIN, 8192)
        lo, hi = max(blk_lo, w_lo), min(blk_hi, w_hi)
        if lo < hi:
            segs.append((lo - blk_lo, hi - blk_lo, w))
    return segs


def _round_bf16(v):
    b = jax.lax.bitcast_convert_type(v, jnp.int32)
    r = (b + jnp.int32(0x7FFF) + ((b >> 16) & jnp.int32(1))) \
        & jnp.int32(-65536)
    return jax.lax.bitcast_convert_type(r, jnp.float32)


def _dist_argmin_kernel(x2_ref, e2_ref, x_ref, em2_ref, idx_ref, wmin_ref,
                        widx_ref):
    n = pl.program_id(1)
    bn = em2_ref.shape[0]
    tok = x_ref.shape[2]
    nb = pl.num_programs(1)
    d = jax.lax.dot_general(
        em2_ref[...], x_ref[0],
        dimension_numbers=(((1,), (0,)), ((), ())),
        preferred_element_type=jnp.float32)
    s = jnp.maximum((x2_ref[0] + e2_ref[...]) + d, 0.0)
    riota = jax.lax.broadcasted_iota(jnp.int32, (bn, tok), 0)
    inf = jnp.float32(jnp.inf)

    @pl.when(n == 0)
    def _():
        wmin_ref[...] = jnp.full(wmin_ref.shape, inf, jnp.float32)
        widx_ref[...] = jnp.zeros(widx_ref.shape, jnp.int32)

    for k in range(_NB):
        @pl.when(n == k)
        def _(k=k):
            for (lo, hi, w) in _segments(k):
                sm = s[lo:hi]
                lmin = jnp.min(sm, axis=0, keepdims=True)
                larg = jnp.min(
                    jnp.where(sm == lmin, riota[lo:hi], jnp.int32(2 ** 30)),
                    axis=0, keepdims=True) + k * bn
                prev = wmin_ref[pl.ds(w, 1), :]
                take = lmin < prev
                wmin_ref[pl.ds(w, 1), :] = jnp.where(take, lmin, prev)
                widx_ref[pl.ds(w, 1), :] = jnp.where(
                    take, larg, widx_ref[pl.ds(w, 1), :])

    @pl.when(n == nb - 1)
    def _():
        acc = _round_bf16(wmin_ref[pl.ds(0, 1), :])
        idx = widx_ref[pl.ds(0, 1), :]
        for w in (1, 2):
            v = wmin_ref[pl.ds(w, 1), :]
            take = v < acc
            acc = jnp.where(take, _round_bf16(v), acc)
            idx = jnp.where(take, widx_ref[pl.ds(w, 1), :], idx)
        idx_ref[0] = idx


def _encode_kernel(idxs_ref, enc_ref, perp_ref, hist_ref):
    b = pl.program_id(0)
    n = pl.program_id(1)
    nbat = pl.num_programs(0)
    nb = pl.num_programs(1)
    tok, bn = enc_ref.shape
    cols = jax.lax.broadcasted_iota(jnp.int32, (tok, bn), 1) + n * bn
    t_oh = (idxs_ref[0] == cols).astype(jnp.float32)
    enc_ref[...] = t_oh
    colsum = jnp.sum(t_oh, axis=0, keepdims=True)

    @pl.when(b == 0)
    def _():
        hist_ref[pl.ds(n, 1), :] = colsum

    @pl.when(b > 0)
    def _():
        hist_ref[pl.ds(n, 1), :] = hist_ref[pl.ds(n, 1), :] + colsum

    @pl.when((b == nbat - 1) & (n == nb - 1))
    def _():
        p = hist_ref[...] * (1.0 / 16384.0)
        plogp = p * jnp.log(p + 1e-10)
        perp_ref[0, 0] = jnp.exp(-jnp.sum(plogp))


def _ste_kernel(q_ref, x_ref, out_ref, loss_ref, lsum_ref):
    b = pl.program_id(0)
    nbat = pl.num_programs(0)
    qt = jnp.transpose(q_ref[0], (1, 0))
    xb = x_ref[0]
    diff = xb - qt
    ss = jnp.sum(diff * diff)

    @pl.when(b == 0)
    def _():
        lsum_ref[0] = ss

    @pl.when(b > 0)
    def _():
        lsum_ref[0] = lsum_ref[0] + ss

    out_ref[0] = xb + (qt - xb)

    @pl.when(b == nbat - 1)
    def _():
        loss_ref[0, 0] = 0.25 * (lsum_ref[0] / (16384.0 * 256.0))


_SC_NC = 2
_SC_NS = 16
_SC_CH = 128


def _sc_gather(codebook, idx_flat):
    nw = _SC_NC * _SC_NS
    bpw = 16384 // nw
    mesh = plsc.VectorSubcoreMesh(core_axis_name="c", subcore_axis_name="s")

    nch = bpw // _SC_CH

    @functools.partial(
        pl.kernel, mesh=mesh,
        out_type=jax.ShapeDtypeStruct((16384, _D), jnp.float32),
        scratch_types=[
            pltpu.VMEM((nch, _SC_CH), jnp.int32),
            pltpu.VMEM((2, _SC_CH, _D), jnp.float32),
            pltpu.SemaphoreType.DMA,
            pltpu.SemaphoreType.DMA,
            pltpu.SemaphoreType.DMA,
            pltpu.SemaphoreType.DMA,
        ],
    )
    def k(table_hbm, idx2_hbm, out_hbm, idx_v, rows_v, gsem0, gsem1,
          osem0, osem1):
        wid = jax.lax.axis_index("s") * _SC_NC + jax.lax.axis_index("c")
        base = wid * bpw
        pltpu.sync_copy(idx2_hbm.at[pl.ds(wid * nch, nch)], idx_v)
        gsems = (gsem0, gsem1)
        osems = (osem0, osem1)
        g_prev = pltpu.async_copy(
            table_hbm.at[idx_v.at[0]], rows_v.at[0], gsems[0])
        out_pending = [None, None]
        for c in range(nch):
            g_prev.wait()
            if c + 1 < nch:
                b = (c + 1) % 2
                if out_pending[b] is not None:
                    out_pending[b].wait()
                    out_pending[b] = None
                g_prev = pltpu.async_copy(
                    table_hbm.at[idx_v.at[c + 1]], rows_v.at[b], gsems[b])
            out_pending[c % 2] = pltpu.async_copy(
                rows_v.at[c % 2],
                out_hbm.at[pl.ds(base + c * _SC_CH, _SC_CH)], osems[c % 2])
        for o in out_pending:
            if o is not None:
                o.wait()

    return k(codebook, idx_flat.reshape(nw * nch, _SC_CH))


def _argmin_call(x2in, e2in, xv, em2):
    return pl.pallas_call(
        _dist_argmin_kernel,
        grid=(_NBATCH, _NB),
        in_specs=[
            pl.BlockSpec((1, 1, _TOK), lambda b, n: (b, 0, 0)),
            pl.BlockSpec((_BN, 1), lambda b, n: (n, 0)),
            pl.BlockSpec((1, _D, _TOK), lambda b, n: (b, 0, 0)),
            pl.BlockSpec((_BN, _D), lambda b, n: (n, 0)),
        ],
        out_specs=pl.BlockSpec((1, 1, _TOK), lambda b, n: (b, 0, 0)),
        out_shape=jax.ShapeDtypeStruct((_NBATCH, 1, _TOK), jnp.int32),
        scratch_shapes=[
            pltpu.VMEM((8, _TOK), jnp.float32),
            pltpu.VMEM((8, _TOK), jnp.int32),
        ],
        compiler_params=pltpu.CompilerParams(
            dimension_semantics=("arbitrary", "arbitrary")),
    )(x2in, e2in, xv, em2)


def _argmin_only(inputs, codebook):
    xv = inputs.reshape(_NBATCH, _D, _TOK)
    flat = jax.lax.optimization_barrier(
        jnp.transpose(inputs, (0, 2, 3, 1)).reshape(-1, _D))
    x2 = jnp.sum(flat ** 2, axis=1)
    e2 = jnp.sum(codebook ** 2, axis=1)
    return _argmin_call(x2.reshape(_NBATCH, 1, _TOK), e2.reshape(8192, 1),
                        xv, -2.0 * codebook)


def _run(inputs, codebook):
    xv = inputs.reshape(_NBATCH, _D, _TOK)
    flat = jax.lax.optimization_barrier(
        jnp.transpose(inputs, (0, 2, 3, 1)).reshape(-1, _D))
    x2 = jnp.sum(flat ** 2, axis=1)
    e2 = jnp.sum(codebook ** 2, axis=1)
    x2in = x2.reshape(_NBATCH, 1, _TOK)
    e2in = e2.reshape(8192, 1)

    idx = _argmin_call(x2in, e2in, xv, -2.0 * codebook)

    qf = _sc_gather(codebook, idx.reshape(16384))
    qn = qf.reshape(_NBATCH, _TOK, _D)

    idxs = idx.reshape(_NBATCH, _TOK, 1)
    enc, perp2d = pl.pallas_call(
        _encode_kernel,
        grid=(_NBATCH, _NB),
        in_specs=[
            pl.BlockSpec((1, _TOK, 1), lambda b, n: (b, 0, 0)),
        ],
        out_specs=[
            pl.BlockSpec((_TOK, _BN), lambda b, n: (b, n)),
            pl.BlockSpec(memory_space=pltpu.SMEM),
        ],
        out_shape=[
            jax.ShapeDtypeStruct((_NBATCH * _TOK, 8192), jnp.float32),
            jax.ShapeDtypeStruct((1, 1), jnp.float32),
        ],
        scratch_shapes=[
            pltpu.VMEM((_NB, _BN), jnp.float32),
        ],
        compiler_params=pltpu.CompilerParams(
            dimension_semantics=("arbitrary", "arbitrary")),
    )(idxs)

    qv, loss2d = pl.pallas_call(
        _ste_kernel,
        grid=(_NBATCH,),
        in_specs=[
            pl.BlockSpec((1, _TOK, _D), lambda b: (b, 0, 0)),
            pl.BlockSpec((1, _D, _TOK), lambda b: (b, 0, 0)),
        ],
        out_specs=[
            pl.BlockSpec((1, _D, _TOK), lambda b: (b, 0, 0)),
            pl.BlockSpec(memory_space=pltpu.SMEM),
        ],
        out_shape=[
            jax.ShapeDtypeStruct((_NBATCH, _D, _TOK), jnp.float32),
            jax.ShapeDtypeStruct((1, 1), jnp.float32),
        ],
        scratch_shapes=[
            pltpu.SMEM((1,), jnp.float32),
        ],
        compiler_params=pltpu.CompilerParams(
            dimension_semantics=("arbitrary",)),
    )(qn, xv)

    quantized = qv.reshape(_NBATCH, _D, 32, 32)
    return (quantized, loss2d.reshape(()), perp2d.reshape(()), enc)


def kernel(inputs, codebook):
    return _run(inputs, codebook)
